# baseline (speedup 1.0000x reference)
import jax
import jax.numpy as jnp
from jax.experimental import pallas as pl
from jax.experimental.pallas import tpu as pltpu

T = 2048
D = 1024
E = 8
K = 2
F = 512
SF = 512


def _moe_kernel(xf_ref, xb_ref, gate_w_ref, gu_ref, down_ref, sgu_ref,
                sdown_ref, out_ref, combine_ref):
    e = pl.program_id(0)
    xb = xb_ref[...]

    @pl.when(e == 0)
    def _router_and_shared():
        xf = xf_ref[...]
        logits = jnp.dot(xf, gate_w_ref[...].T, preferred_element_type=jnp.float32)
        logits = logits - jnp.max(logits, axis=-1, keepdims=True)
        ex = jnp.exp(logits)
        probs = ex / jnp.sum(ex, axis=-1, keepdims=True)
        a1 = jnp.argmax(probs, axis=-1)
        m1 = jnp.max(probs, axis=-1)
        col = jax.lax.broadcasted_iota(jnp.int32, (T, E), 1)
        masked = jnp.where(col == a1[:, None], -jnp.inf, probs)
        a2 = jnp.argmax(masked, axis=-1)
        m2 = jnp.max(masked, axis=-1)
        s = m1 + m2
        w1 = (m1 / s)[:, None]
        w2 = (m2 / s)[:, None]
        combine_ref[...] = jnp.where(col == a1[:, None], w1, 0.0) + jnp.where(
            col == a2[:, None], w2, 0.0)

        sh = jnp.dot(xb, sgu_ref[...], preferred_element_type=jnp.float32)
        sg = sh[:, :SF]
        su = sh[:, SF:]
        act = ((sg * jax.nn.sigmoid(sg)) * su).astype(jnp.bfloat16)
        out_ref[...] = jnp.dot(act, sdown_ref[...], preferred_element_type=jnp.float32)

    h = jnp.dot(xb, gu_ref[0], preferred_element_type=jnp.float32)
    g = h[:, :F]
    u = h[:, F:]
    act = ((g * jax.nn.sigmoid(g)) * u).astype(jnp.bfloat16)
    eo = jnp.dot(act, down_ref[0], preferred_element_type=jnp.float32)
    col = jax.lax.broadcasted_iota(jnp.int32, (T, E), 1)
    w = jnp.sum(jnp.where(col == e, combine_ref[...], 0.0), axis=-1,
                keepdims=True)
    out_ref[...] += w * eo


@jax.jit
def kernel(hidden_states, gate_w, expert_gate_up, expert_down, shared_gate_up,
           shared_down):
    xb = hidden_states.astype(jnp.bfloat16)
    gu = expert_gate_up.astype(jnp.bfloat16)
    dn = expert_down.astype(jnp.bfloat16)
    sgu = shared_gate_up.astype(jnp.bfloat16)
    sdn = shared_down.astype(jnp.bfloat16)
    return pl.pallas_call(
        _moe_kernel,
        grid=(E,),
        in_specs=[
            pl.BlockSpec((T, D), lambda e: (0, 0)),
            pl.BlockSpec((T, D), lambda e: (0, 0)),
            pl.BlockSpec((E, D), lambda e: (0, 0)),
            pl.BlockSpec((1, D, 2 * F), lambda e: (e, 0, 0)),
            pl.BlockSpec((1, F, D), lambda e: (e, 0, 0)),
            pl.BlockSpec((D, 2 * SF), lambda e: (0, 0)),
            pl.BlockSpec((SF, D), lambda e: (0, 0)),
        ],
        out_specs=pl.BlockSpec((T, D), lambda e: (0, 0)),
        out_shape=jax.ShapeDtypeStruct((T, D), jnp.float32),
        scratch_shapes=[pltpu.VMEM((T, E), jnp.float32)],
        compiler_params=pltpu.CompilerParams(
            dimension_semantics=("arbitrary",),
            vmem_limit_bytes=100 * 1024 * 1024,
        ),
    )(hidden_states, xb, gate_w, gu, dn, sgu, sdn)
